# baseline (device time: 743040 ns/iter reference)
import jax
import jax.numpy as jnp
from jax import lax
from jax.experimental import pallas as pl
from jax.experimental.pallas import tpu as pltpu

N = 32
M = 64
D = 512
NLAYER = 3


def kernel(x, Win0, Wout0, Win1, Wout1, Win2, Wout2):
    def body(x_ref, win0, wout0, win1, wout1, win2, wout2, out_ref,
             xfull, part, land, send_sem, ag_sems, rs_sems):
        me = lax.axis_index("i")
        left = (me + N - 1) % N
        right = (me + 1) % N

        barrier = pltpu.get_barrier_semaphore()
        for nbr in (left, right):
            pl.semaphore_signal(barrier, inc=1, device_id=(nbr,),
                                device_id_type=pl.DeviceIdType.MESH)
        pl.semaphore_wait(barrier, 2)

        def ring_ag(own):
            for h in range(N - 1):
                c = (own - h) % N
                rdma = pltpu.make_async_remote_copy(
                    src_ref=xfull.at[c],
                    dst_ref=xfull.at[c],
                    send_sem=send_sem,
                    recv_sem=ag_sems.at[h],
                    device_id=(right,),
                    device_id_type=pl.DeviceIdType.MESH,
                )
                rdma.start()
                rdma.wait()

        def ring_rs():
            for s in range(N - 1):
                c = (me - s) % N
                c_next = (me - s - 1) % N
                rdma = pltpu.make_async_remote_copy(
                    src_ref=part.at[c],
                    dst_ref=land.at[s],
                    send_sem=send_sem,
                    recv_sem=rs_sems.at[s],
                    device_id=(right,),
                    device_id_type=pl.DeviceIdType.MESH,
                )
                rdma.start()
                rdma.wait()
                part[c_next] = part[c_next] + land[s]

        def layer(win, wout):
            xb = xfull[...].reshape(N * M, D).astype(jnp.bfloat16)
            h = jnp.dot(xb, win[...].astype(jnp.bfloat16),
                        preferred_element_type=jnp.float32)
            h = jnp.maximum(h, 0.0).astype(jnp.bfloat16)
            p = jnp.dot(h, wout[...].astype(jnp.bfloat16),
                        preferred_element_type=jnp.float32)
            part[...] = p.reshape(N, M, D)

        xfull[me] = x_ref[...]
        ring_ag(me)

        own = (me + 1) % N
        for win, wout in ((win0, wout0), (win1, wout1), (win2, wout2)):
            layer(win, wout)
            ring_rs()
            xfull[own] = part[own]
            ring_ag(own)

        out_ref[...] = xfull[...].reshape(N * M, D)

    return pl.pallas_call(
        body,
        out_shape=jax.ShapeDtypeStruct((N * M, D), jnp.float32),
        in_specs=[pl.BlockSpec(memory_space=pltpu.VMEM)] * 7,
        out_specs=pl.BlockSpec(memory_space=pltpu.VMEM),
        scratch_shapes=[
            pltpu.VMEM((N, M, D), jnp.float32),
            pltpu.VMEM((N, M, D), jnp.float32),
            pltpu.VMEM((N - 1, M, D), jnp.float32),
            pltpu.SemaphoreType.DMA,
            pltpu.SemaphoreType.DMA((N - 1,)),
            pltpu.SemaphoreType.DMA((N - 1,)),
        ],
        compiler_params=pltpu.CompilerParams(collective_id=0),
    )(x, Win0, Wout0, Win1, Wout1, Win2, Wout2)


# device time: 263174 ns/iter; 2.8234x vs baseline; 2.8234x over previous
import functools

import jax
import jax.numpy as jnp
from jax import lax
from jax.experimental import pallas as pl
from jax.experimental.pallas import tpu as pltpu

N = 32
M = 64
D = 512
B_ORDER = (0, 3, 1, 2, 4)

PERM = [0] * N
for _g in range(N):
    _q = 0
    for _k, _b in enumerate(B_ORDER):
        _q |= ((_g >> _b) & 1) << (4 - _k)
    PERM[_g] = _q

LAND_OFF = (0, 16, 24, 28, 30)


def kernel(x, Win0, Wout0, Win1, Wout1, Win2, Wout2):
    def body(x_ref, win0, wout0, win1, wout1, win2, wout2, out_ref,
             xq, part, land, sendbuf, send_sem, ag_sems, rs_sems):
        me = lax.axis_index("i")
        partners = [me ^ (1 << b) for b in range(5)]

        q_me = ((me & 1) << 4) | (((me >> 3) & 1) << 3) | (((me >> 1) & 1) << 2) \
            | (((me >> 2) & 1) << 1) | ((me >> 4) & 1)

        barrier = pltpu.get_barrier_semaphore()
        for p in partners:
            pl.semaphore_signal(barrier, inc=1, device_id=(p,),
                                device_id_type=pl.DeviceIdType.MESH)
        pl.semaphore_wait(barrier, 5)

        def all_gather():
            a = q_me
            s = 1
            for j, b in enumerate(reversed(B_ORDER)):
                partner = me ^ (1 << b)
                rdma = pltpu.make_async_remote_copy(
                    src_ref=xq.at[pl.ds(a, s)],
                    dst_ref=xq.at[pl.ds(a, s)],
                    send_sem=send_sem,
                    recv_sem=ag_sems.at[j],
                    device_id=(partner,),
                    device_id_type=pl.DeviceIdType.MESH,
                )
                rdma.start()
                rdma.wait()
                a = a - (a & s)
                s *= 2

        def reduce_scatter():
            a = jnp.int32(0)
            s = N
            for k, b in enumerate(B_ORDER):
                partner = me ^ (1 << b)
                half = s // 2
                bit = (me >> b) & 1
                send_start = a + (1 - bit) * half
                sendbuf[pl.ds(0, half)] = part[pl.ds(send_start, half)].astype(
                    jnp.bfloat16)
                rdma = pltpu.make_async_remote_copy(
                    src_ref=sendbuf.at[pl.ds(0, half)],
                    dst_ref=land.at[pl.ds(LAND_OFF[k], half)],
                    send_sem=send_sem,
                    recv_sem=rs_sems.at[k],
                    device_id=(partner,),
                    device_id_type=pl.DeviceIdType.MESH,
                )
                rdma.start()
                rdma.wait()
                a = a + bit * half
                part[pl.ds(a, half)] = part[pl.ds(a, half)] + land[
                    pl.ds(LAND_OFF[k], half)].astype(jnp.float32)
                s = half

        def layer(win, wout):
            xb = xq[...].reshape(N * M, D)
            h = jnp.dot(xb, win[...].astype(jnp.bfloat16),
                        preferred_element_type=jnp.float32)
            h = jnp.maximum(h, 0.0).astype(jnp.bfloat16)
            p = jnp.dot(h, wout[...].astype(jnp.bfloat16),
                        preferred_element_type=jnp.float32)
            part[...] = p.reshape(N, M, D)

        xq[q_me] = x_ref[...].astype(jnp.bfloat16)
        all_gather()

        for win, wout in ((win0, wout0), (win1, wout1), (win2, wout2)):
            layer(win, wout)
            reduce_scatter()
            xq[q_me] = part[q_me].astype(jnp.bfloat16)
            all_gather()

        for g in range(N):
            out_ref[pl.ds(g * M, M), :] = xq[PERM[g]].astype(jnp.float32)

        @functools.partial(pl.run_scoped,
                           exit_sem=pltpu.SemaphoreType.REGULAR)
        def _(exit_sem):
            for p in partners:
                pl.semaphore_signal(exit_sem, inc=1, device_id=(p,),
                                    device_id_type=pl.DeviceIdType.MESH)
            pl.semaphore_wait(exit_sem, 5)

    return pl.pallas_call(
        body,
        out_shape=jax.ShapeDtypeStruct((N * M, D), jnp.float32),
        in_specs=[pl.BlockSpec(memory_space=pltpu.VMEM)] * 7,
        out_specs=pl.BlockSpec(memory_space=pltpu.VMEM),
        scratch_shapes=[
            pltpu.VMEM((N, M, D), jnp.bfloat16),
            pltpu.VMEM((N, M, D), jnp.float32),
            pltpu.VMEM((N, M, D), jnp.bfloat16),
            pltpu.VMEM((N // 2, M, D), jnp.bfloat16),
            pltpu.SemaphoreType.DMA,
            pltpu.SemaphoreType.DMA((5,)),
            pltpu.SemaphoreType.DMA((5,)),
        ],
        compiler_params=pltpu.CompilerParams(collective_id=0),
    )(x, Win0, Wout0, Win1, Wout1, Win2, Wout2)


# device time: 223845 ns/iter; 3.3194x vs baseline; 1.1757x over previous
import functools

import jax
import jax.numpy as jnp
from jax import lax
from jax.experimental import pallas as pl
from jax.experimental.pallas import tpu as pltpu

N = 32
M = 64
D = 512

_BIT_HOPS = (1, 2, 2, 1, 2)


def _est_hops(j):
    return sum(_BIT_HOPS[b] for b in range(5) if (j >> b) & 1)


J_ORDER = sorted(range(1, N), key=lambda j: -_est_hops(j))


def kernel(x, Win0, Wout0, Win1, Wout1, Win2, Wout2):
    def body(x_ref, win0, wout0, win1, wout1, win2, wout2, out_ref,
             xq, part, land, sendb, send_sems, rs_recv_sem, ag_recv_sem):
        me = lax.axis_index("i")

        barrier = pltpu.get_barrier_semaphore()
        for j in range(1, N):
            pl.semaphore_signal(barrier, inc=1, device_id=(me ^ j,),
                                device_id_type=pl.DeviceIdType.MESH)
        pl.semaphore_wait(barrier, N - 1)

        def drain(descs):
            for d in descs:
                d.wait_send()

        def bcast_own_chunk():
            descs = []
            for j in J_ORDER:
                tgt = me ^ j
                rdma = pltpu.make_async_remote_copy(
                    src_ref=xq.at[me],
                    dst_ref=xq.at[me],
                    send_sem=send_sems.at[J_ORDER.index(j)],
                    recv_sem=ag_recv_sem,
                    device_id=(tgt,),
                    device_id_type=pl.DeviceIdType.MESH,
                )
                rdma.start()
                descs.append(rdma)
            for _ in range(N - 1):
                descs[0].wait_recv()
            drain(descs)

        def a2a_reduce():
            sendb[...] = part[...].astype(jnp.bfloat16)
            land[me] = jnp.zeros((M, D), jnp.bfloat16)
            descs = []
            for j in J_ORDER:
                tgt = me ^ j
                rdma = pltpu.make_async_remote_copy(
                    src_ref=sendb.at[tgt],
                    dst_ref=land.at[me],
                    send_sem=send_sems.at[J_ORDER.index(j)],
                    recv_sem=rs_recv_sem,
                    device_id=(tgt,),
                    device_id_type=pl.DeviceIdType.MESH,
                )
                rdma.start()
                descs.append(rdma)
            for _ in range(N - 1):
                descs[0].wait_recv()
            red = part[me] + jnp.sum(land[...].astype(jnp.float32), axis=0)
            xq[me] = red.astype(jnp.bfloat16)
            drain(descs)

        def layer(win, wout):
            xb = xq[...].reshape(N * M, D)
            h = jnp.dot(xb, win[...].astype(jnp.bfloat16),
                        preferred_element_type=jnp.float32)
            h = jnp.maximum(h, 0.0).astype(jnp.bfloat16)
            p = jnp.dot(h, wout[...].astype(jnp.bfloat16),
                        preferred_element_type=jnp.float32)
            part[...] = p.reshape(N, M, D)

        xq[me] = x_ref[...].astype(jnp.bfloat16)
        bcast_own_chunk()

        for win, wout in ((win0, wout0), (win1, wout1), (win2, wout2)):
            layer(win, wout)
            a2a_reduce()
            bcast_own_chunk()

        out_ref[...] = xq[...].reshape(N * M, D)

        @functools.partial(pl.run_scoped,
                           exit_sem=pltpu.SemaphoreType.REGULAR)
        def _(exit_sem):
            for j in range(1, N):
                pl.semaphore_signal(exit_sem, inc=1, device_id=(me ^ j,),
                                    device_id_type=pl.DeviceIdType.MESH)
            pl.semaphore_wait(exit_sem, N - 1)

    return pl.pallas_call(
        body,
        out_shape=jax.ShapeDtypeStruct((N * M, D), jnp.bfloat16),
        in_specs=[pl.BlockSpec(memory_space=pltpu.VMEM)] * 7,
        out_specs=pl.BlockSpec(memory_space=pltpu.VMEM),
        scratch_shapes=[
            pltpu.VMEM((N, M, D), jnp.bfloat16),
            pltpu.VMEM((N, M, D), jnp.float32),
            pltpu.VMEM((N, M, D), jnp.bfloat16),
            pltpu.VMEM((N, M, D), jnp.bfloat16),
            pltpu.SemaphoreType.DMA((N - 1,)),
            pltpu.SemaphoreType.DMA,
            pltpu.SemaphoreType.DMA,
        ],
        compiler_params=pltpu.CompilerParams(collective_id=0),
    )(x, Win0, Wout0, Win1, Wout1, Win2, Wout2)


# device time: 223655 ns/iter; 3.3223x vs baseline; 1.0008x over previous
import functools

import jax
import jax.numpy as jnp
from jax import lax
from jax.experimental import pallas as pl
from jax.experimental.pallas import tpu as pltpu

N = 32
M = 64
D = 512

_BIT_HOPS = (1, 2, 2, 1, 2)


def _est_hops(j):
    return sum(_BIT_HOPS[b] for b in range(5) if (j >> b) & 1)


J_ORDER = sorted(range(1, N), key=lambda j: -_est_hops(j))


def kernel(x, Win0, Wout0, Win1, Wout1, Win2, Wout2):
    def body(x_ref, win0, wout0, win1, wout1, win2, wout2, out_ref,
             xq, part, land, sendb, send_sems, rs_recv_sem, ag_recv_sem):
        me = lax.axis_index("i")

        barrier = pltpu.get_barrier_semaphore()
        for j in range(1, N):
            pl.semaphore_signal(barrier, inc=1, device_id=(me ^ j,),
                                device_id_type=pl.DeviceIdType.MESH)
        pl.semaphore_wait(barrier, N - 1)

        def drain(descs):
            for d in descs:
                d.wait_send()

        def bcast_own_chunk():
            descs = []
            for j in J_ORDER:
                tgt = me ^ j
                rdma = pltpu.make_async_remote_copy(
                    src_ref=xq.at[me],
                    dst_ref=xq.at[me],
                    send_sem=send_sems.at[J_ORDER.index(j)],
                    recv_sem=ag_recv_sem,
                    device_id=(tgt,),
                    device_id_type=pl.DeviceIdType.MESH,
                )
                rdma.start()
                descs.append(rdma)
            for _ in range(N - 1):
                descs[0].wait_recv()
            drain(descs)

        def a2a_reduce():
            descs = []
            for j in J_ORDER:
                tgt = me ^ j
                rdma = pltpu.make_async_remote_copy(
                    src_ref=sendb.at[tgt],
                    dst_ref=land.at[me],
                    send_sem=send_sems.at[J_ORDER.index(j)],
                    recv_sem=rs_recv_sem,
                    device_id=(tgt,),
                    device_id_type=pl.DeviceIdType.MESH,
                )
                rdma.start()
                descs.append(rdma)
            for _ in range(N - 1):
                descs[0].wait_recv()
            red = part[me] + jnp.sum(land[...].astype(jnp.float32), axis=0)
            xq[me] = red.astype(jnp.bfloat16)
            drain(descs)

        def layer(win, wout):
            xb = xq[...].reshape(N * M, D)
            h = jnp.dot(xb, win[...].astype(jnp.bfloat16),
                        preferred_element_type=jnp.float32)
            h = jnp.maximum(h, 0.0).astype(jnp.bfloat16)
            p = jnp.dot(h, wout[...].astype(jnp.bfloat16),
                        preferred_element_type=jnp.float32)
            p3 = p.reshape(N, M, D)
            sendb[...] = p3.astype(jnp.bfloat16)
            part[...] = p3

        land[me] = jnp.zeros((M, D), jnp.bfloat16)
        xq[me] = x_ref[...].astype(jnp.bfloat16)
        bcast_own_chunk()

        for win, wout in ((win0, wout0), (win1, wout1), (win2, wout2)):
            layer(win, wout)
            a2a_reduce()
            bcast_own_chunk()

        out_ref[...] = xq[...].reshape(N * M, D)

        @functools.partial(pl.run_scoped,
                           exit_sem=pltpu.SemaphoreType.REGULAR)
        def _(exit_sem):
            for j in range(1, N):
                pl.semaphore_signal(exit_sem, inc=1, device_id=(me ^ j,),
                                    device_id_type=pl.DeviceIdType.MESH)
            pl.semaphore_wait(exit_sem, N - 1)

    return pl.pallas_call(
        body,
        out_shape=jax.ShapeDtypeStruct((N * M, D), jnp.bfloat16),
        in_specs=[pl.BlockSpec(memory_space=pltpu.VMEM)] * 7,
        out_specs=pl.BlockSpec(memory_space=pltpu.VMEM),
        scratch_shapes=[
            pltpu.VMEM((N, M, D), jnp.bfloat16),
            pltpu.VMEM((N, M, D), jnp.float32),
            pltpu.VMEM((N, M, D), jnp.bfloat16),
            pltpu.VMEM((N, M, D), jnp.bfloat16),
            pltpu.SemaphoreType.DMA((N - 1,)),
            pltpu.SemaphoreType.DMA,
            pltpu.SemaphoreType.DMA,
        ],
        compiler_params=pltpu.CompilerParams(collective_id=0),
    )(x, Win0, Wout0, Win1, Wout1, Win2, Wout2)
